# Initial kernel scaffold; baseline (speedup 1.0000x reference)
import jax
import jax.numpy as jnp
from jax.experimental import pallas as pl


def kernel(x, edge_index, W, b):
    raise NotImplementedError("write your pallas kernel here")


# speedup vs baseline: 21.9432x; 21.9432x over previous
import functools

import jax
import jax.numpy as jnp
from jax import lax
from jax.experimental import pallas as pl
from jax.experimental.pallas import tpu as pltpu
from jax.experimental.pallas import tpu_sc as plsc

N = 10000
E = 320000
D = 128

NC = 2
NS = 16
NW = NC * NS

BATCH = 128
NPAD = 10240
ROWS_PER_TILE = NPAD // NS
EPW = ((E // NW + BATCH - 1) // BATCH) * BATCH
NB = EPW // BATCH
EPAD = EPW * NW

_MESH = plsc.VectorSubcoreMesh(
    core_axis_name="c", subcore_axis_name="s", num_cores=NC, num_subcores=NS
)


@functools.partial(
    pl.kernel,
    out_type=jax.ShapeDtypeStruct((NC * NPAD,), jnp.float32),
    mesh=_MESH,
    scratch_types=[
        pltpu.VMEM((BATCH,), jnp.int32),
        pltpu.VMEM((BATCH,), jnp.float32),
        pltpu.VMEM((ROWS_PER_TILE,), jnp.float32),
        pltpu.VMEM_SHARED((NPAD,), jnp.float32),
    ],
)
def _sc_deg(dst_hbm, deg_out_hbm, idx_v, ones_v, degbuf_v, deg_sh):
    cid = lax.axis_index("c")
    sid = lax.axis_index("s")

    for j in range(ROWS_PER_TILE // 16):
        degbuf_v[pl.ds(16 * j, 16)] = jnp.zeros((16,), jnp.float32)
    pltpu.sync_copy(degbuf_v, deg_sh.at[pl.ds(sid * ROWS_PER_TILE, ROWS_PER_TILE)])
    for j in range(BATCH // 16):
        ones_v[pl.ds(16 * j, 16)] = jnp.ones((16,), jnp.float32)
    plsc.subcore_barrier()

    wid = cid * NS + sid

    def body(i, carry):
        base = wid * EPW + i * BATCH
        pltpu.sync_copy(dst_hbm.at[pl.ds(base, BATCH)], idx_v)
        pltpu.sync_copy(ones_v, deg_sh.at[idx_v], add=True)
        return carry

    lax.fori_loop(0, NB, body, 0)
    plsc.subcore_barrier()

    r0 = sid * ROWS_PER_TILE
    pltpu.sync_copy(deg_sh.at[pl.ds(r0, ROWS_PER_TILE)], degbuf_v)
    pltpu.sync_copy(degbuf_v, deg_out_hbm.at[pl.ds(cid * NPAD + r0, ROWS_PER_TILE)])


@functools.partial(
    pl.kernel,
    out_type=jax.ShapeDtypeStruct((NC * NPAD, D), jnp.float32),
    mesh=_MESH,
    scratch_types=[
        pltpu.VMEM((BATCH,), jnp.int32),
        pltpu.VMEM((BATCH,), jnp.int32),
        pltpu.VMEM((BATCH, D), jnp.float32),
        pltpu.VMEM_SHARED((NPAD, D), jnp.float32),
        pltpu.SemaphoreType.DMA,
    ],
)
def _sc_msg(g_hbm, src_hbm, dst_hbm, acc_out_hbm, src_v, dst_v, rows_v, acc_sh, sem):
    cid = lax.axis_index("c")
    sid = lax.axis_index("s")

    for k in range(ROWS_PER_TILE // BATCH):
        r0 = sid * ROWS_PER_TILE + k * BATCH
        pltpu.sync_copy(g_hbm.at[pl.ds(r0, BATCH)], rows_v)
        pltpu.sync_copy(rows_v, acc_sh.at[pl.ds(r0, BATCH)])
    plsc.subcore_barrier()

    wid = cid * NS + sid

    def body(i, carry):
        base = wid * EPW + i * BATCH
        pltpu.sync_copy(src_hbm.at[pl.ds(base, BATCH)], src_v)
        pltpu.sync_copy(dst_hbm.at[pl.ds(base, BATCH)], dst_v)
        pltpu.async_copy(g_hbm.at[src_v], rows_v, sem).wait()
        pltpu.sync_copy(rows_v, acc_sh.at[dst_v], add=True)
        return carry

    lax.fori_loop(0, NB, body, 0)
    plsc.subcore_barrier()

    for k in range(ROWS_PER_TILE // BATCH):
        r0 = sid * ROWS_PER_TILE + k * BATCH
        pltpu.sync_copy(acc_sh.at[pl.ds(r0, BATCH)], rows_v)
        pltpu.sync_copy(rows_v, acc_out_hbm.at[pl.ds(cid * NPAD + r0, BATCH)])


_RB = 1280


def _tc_g_body(x_ref, w_ref, deg_ref, g_ref):
    h = jnp.dot(x_ref[...], w_ref[...], preferred_element_type=jnp.float32)
    dis = lax.rsqrt(deg_ref[0, :] + deg_ref[1, :] + 1.0)
    g_ref[...] = h * dis[:, None]


def _tc_g(x_pad, w, deg2):
    return pl.pallas_call(
        _tc_g_body,
        grid=(NPAD // _RB,),
        in_specs=[
            pl.BlockSpec((_RB, D), lambda i: (i, 0)),
            pl.BlockSpec((D, D), lambda i: (0, 0)),
            pl.BlockSpec((NC, _RB), lambda i: (0, i)),
        ],
        out_specs=pl.BlockSpec((_RB, D), lambda i: (i, 0)),
        out_shape=jax.ShapeDtypeStruct((NPAD, D), jnp.float32),
    )(x_pad, w, deg2)


def _tc_out_body(a_ref, g_ref, deg_ref, b_ref, o_ref):
    dis = lax.rsqrt(deg_ref[0, :] + deg_ref[1, :] + 1.0)
    s = a_ref[0] + a_ref[1] - g_ref[...]
    o_ref[...] = jnp.maximum(dis[:, None] * s + b_ref[...], 0.0)


def _tc_out(acc2, g, deg2, b2d):
    return pl.pallas_call(
        _tc_out_body,
        grid=(NPAD // _RB,),
        in_specs=[
            pl.BlockSpec((NC, _RB, D), lambda i: (0, i, 0)),
            pl.BlockSpec((_RB, D), lambda i: (i, 0)),
            pl.BlockSpec((NC, _RB), lambda i: (0, i)),
            pl.BlockSpec((1, D), lambda i: (0, 0)),
        ],
        out_specs=pl.BlockSpec((_RB, D), lambda i: (i, 0)),
        out_shape=jax.ShapeDtypeStruct((NPAD, D), jnp.float32),
    )(acc2, g, deg2, b2d)


def kernel(x, edge_index, W, b):
    src = edge_index[0].astype(jnp.int32)
    dst = edge_index[1].astype(jnp.int32)
    pad_idx = N + jnp.arange(EPAD - E, dtype=jnp.int32) % (NPAD - N)
    srcp = jnp.concatenate([src, pad_idx])
    dstp = jnp.concatenate([dst, pad_idx])
    x_pad = jnp.zeros((NPAD, D), jnp.float32).at[:N].set(x)

    deg2 = _sc_deg(dstp).reshape(NC, NPAD)
    g = _tc_g(x_pad, W, deg2)
    acc2 = _sc_msg(g, srcp, dstp).reshape(NC, NPAD, D)
    out = _tc_out(acc2, g, deg2, b.reshape(1, D))
    return out[:N]


# speedup vs baseline: 42.9789x; 1.9586x over previous
import functools

import jax
import jax.numpy as jnp
from jax import lax
from jax.experimental import pallas as pl
from jax.experimental.pallas import tpu as pltpu
from jax.experimental.pallas import tpu_sc as plsc

N = 10000
E = 320000
D = 128

NC = 2
NS = 16
NW = NC * NS

BATCH = 128
NPAD = 10240
ROWS_PER_TILE = NPAD // NS
NB = 80
EPW = NB * BATCH
EPAD = EPW * NW
NSB = NB // 2

_MESH = plsc.VectorSubcoreMesh(
    core_axis_name="c", subcore_axis_name="s", num_cores=NC, num_subcores=NS
)


@functools.partial(
    pl.kernel,
    out_type=jax.ShapeDtypeStruct((NC * NPAD,), jnp.float32),
    mesh=_MESH,
    scratch_types=[
        pltpu.VMEM((NB, BATCH), jnp.int32),
        pltpu.VMEM((BATCH,), jnp.float32),
        pltpu.VMEM((ROWS_PER_TILE,), jnp.float32),
        pltpu.VMEM_SHARED((NPAD,), jnp.float32),
        pltpu.SemaphoreType.DMA,
    ],
)
def _sc_deg(dst_hbm, deg_out_hbm, idx_v, ones_v, degbuf_v, deg_sh, sem):
    cid = lax.axis_index("c")
    sid = lax.axis_index("s")
    wid = cid * NS + sid

    for j in range(ROWS_PER_TILE // 16):
        degbuf_v[pl.ds(16 * j, 16)] = jnp.zeros((16,), jnp.float32)
    pltpu.sync_copy(degbuf_v, deg_sh.at[pl.ds(sid * ROWS_PER_TILE, ROWS_PER_TILE)])
    for j in range(BATCH // 16):
        ones_v[pl.ds(16 * j, 16)] = jnp.ones((16,), jnp.float32)
    pltpu.sync_copy(dst_hbm.at[wid], idx_v)
    plsc.subcore_barrier()

    fired = [
        pltpu.async_copy(ones_v, deg_sh.at[idx_v.at[i]], sem, add=True)
        for i in range(NB)
    ]
    for d in fired:
        d.wait()
    plsc.subcore_barrier()

    r0 = sid * ROWS_PER_TILE
    pltpu.sync_copy(deg_sh.at[pl.ds(r0, ROWS_PER_TILE)], degbuf_v)
    pltpu.sync_copy(degbuf_v, deg_out_hbm.at[pl.ds(cid * NPAD + r0, ROWS_PER_TILE)])


WIN = 40
NWIN = NB // WIN


@functools.partial(
    pl.kernel,
    out_type=jax.ShapeDtypeStruct((NC * NPAD, D), jnp.float32),
    mesh=_MESH,
    scratch_types=[
        pltpu.VMEM((WIN, BATCH), jnp.int32),
        pltpu.VMEM((WIN, BATCH), jnp.int32),
        pltpu.VMEM((BATCH, D), jnp.float32),
        pltpu.VMEM((BATCH, D), jnp.float32),
        pltpu.VMEM_SHARED((NPAD, D), jnp.float32),
        pltpu.SemaphoreType.DMA,
        pltpu.SemaphoreType.DMA,
        pltpu.SemaphoreType.DMA,
    ],
)
def _sc_msg(g_hbm, src_hbm, dst_hbm, acc_out_hbm,
            src_v, dst_v, buf_a, buf_b, acc_sh, sem_a, sem_b, sem_s):
    cid = lax.axis_index("c")
    sid = lax.axis_index("s")
    wid = cid * NS + sid

    for k in range(ROWS_PER_TILE // BATCH):
        r0 = sid * ROWS_PER_TILE + k * BATCH
        pltpu.sync_copy(g_hbm.at[pl.ds(r0, BATCH)], buf_a)
        pltpu.sync_copy(buf_a, acc_sh.at[pl.ds(r0, BATCH)])
    plsc.subcore_barrier()

    def issue(i, buf, sem):
        pltpu.async_copy(g_hbm.at[src_v.at[i]], buf, sem)

    def drain(i, buf, sem):
        pltpu.make_async_copy(g_hbm.at[src_v.at[i]], buf, sem).wait()

    def scatter(i, buf):
        pltpu.async_copy(buf, acc_sh.at[dst_v.at[i]], sem_s, add=True).wait()

    for w in range(NWIN):
        pltpu.sync_copy(src_hbm.at[wid, pl.ds(w * WIN, WIN)], src_v)
        pltpu.sync_copy(dst_hbm.at[wid, pl.ds(w * WIN, WIN)], dst_v)
        issue(0, buf_a, sem_a)

        def body(j, carry):
            i = 2 * j
            issue(i + 1, buf_b, sem_b)
            drain(i, buf_a, sem_a)
            scatter(i, buf_a)
            issue(i + 2, buf_a, sem_a)
            drain(i + 1, buf_b, sem_b)
            scatter(i + 1, buf_b)
            return carry

        lax.fori_loop(0, WIN // 2 - 1, body, 0)
        i = WIN - 2
        issue(i + 1, buf_b, sem_b)
        drain(i, buf_a, sem_a)
        scatter(i, buf_a)
        drain(i + 1, buf_b, sem_b)
        scatter(i + 1, buf_b)
    plsc.subcore_barrier()

    for k in range(ROWS_PER_TILE // BATCH):
        r0 = sid * ROWS_PER_TILE + k * BATCH
        pltpu.sync_copy(acc_sh.at[pl.ds(r0, BATCH)], buf_a)
        pltpu.sync_copy(buf_a, acc_out_hbm.at[pl.ds(cid * NPAD + r0, BATCH)])


_RB = 1280


def _tc_g_body(x_ref, w_ref, deg_ref, g_ref):
    h = jnp.dot(x_ref[...], w_ref[...], preferred_element_type=jnp.float32)
    dis = lax.rsqrt(deg_ref[0, :] + deg_ref[1, :] + 1.0)
    g_ref[...] = h * dis[:, None]


def _tc_g(x_pad, w, deg2):
    return pl.pallas_call(
        _tc_g_body,
        grid=(NPAD // _RB,),
        in_specs=[
            pl.BlockSpec((_RB, D), lambda i: (i, 0)),
            pl.BlockSpec((D, D), lambda i: (0, 0)),
            pl.BlockSpec((NC, _RB), lambda i: (0, i)),
        ],
        out_specs=pl.BlockSpec((_RB, D), lambda i: (i, 0)),
        out_shape=jax.ShapeDtypeStruct((NPAD, D), jnp.float32),
    )(x_pad, w, deg2)


def _tc_out_body(a_ref, g_ref, deg_ref, b_ref, o_ref):
    dis = lax.rsqrt(deg_ref[0, :] + deg_ref[1, :] + 1.0)
    s = a_ref[0] + a_ref[1] - g_ref[...]
    o_ref[...] = jnp.maximum(dis[:, None] * s + b_ref[...], 0.0)


def _tc_out(acc2, g, deg2, b2d):
    return pl.pallas_call(
        _tc_out_body,
        grid=(NPAD // _RB,),
        in_specs=[
            pl.BlockSpec((NC, _RB, D), lambda i: (0, i, 0)),
            pl.BlockSpec((_RB, D), lambda i: (i, 0)),
            pl.BlockSpec((NC, _RB), lambda i: (0, i)),
            pl.BlockSpec((1, D), lambda i: (0, 0)),
        ],
        out_specs=pl.BlockSpec((_RB, D), lambda i: (i, 0)),
        out_shape=jax.ShapeDtypeStruct((NPAD, D), jnp.float32),
    )(acc2, g, deg2, b2d)


def kernel(x, edge_index, W, b):
    src = edge_index[0].astype(jnp.int32)
    dst = edge_index[1].astype(jnp.int32)
    pad_idx = N + jnp.arange(EPAD - E, dtype=jnp.int32) % (NPAD - N)
    srcp = jnp.concatenate([src, pad_idx]).reshape(NW, NB, BATCH)
    dstp = jnp.concatenate([dst, pad_idx]).reshape(NW, NB, BATCH)
    x_pad = jnp.zeros((NPAD, D), jnp.float32).at[:N].set(x)

    deg2 = _sc_deg(dstp).reshape(NC, NPAD)
    g = _tc_g(x_pad, W, deg2)
    acc2 = _sc_msg(g, srcp, dstp).reshape(NC, NPAD, D)
    out = _tc_out(acc2, g, deg2, b.reshape(1, D))
    return out[:N]
